# Initial kernel scaffold; baseline (speedup 1.0000x reference)
import jax
import jax.numpy as jnp
from jax.experimental import pallas as pl


def kernel(z, W):
    raise NotImplementedError("write your pallas kernel here")


# speedup vs baseline: 1.5237x; 1.5237x over previous
import functools

import jax
import jax.numpy as jnp
from jax.experimental import pallas as pl
from jax.experimental.pallas import tpu as pltpu

CODEBOOK = 1024
EMB = 256
B = 8
TOK = 1024
BETA = 0.25
N_TOKENS = B * TOK
N_ELEMS = N_TOKENS * EMB


def _vq_body(z_ref, w_ref, wt_ref,
             zq_ref, idx_ref, loss_ref, perp_ref, md_ref,
             dsum_acc, sqsum_acc, counts_acc):
    b = pl.program_id(0)
    zb = z_ref[0]
    w = w_ref[...]

    m = jax.lax.dot_general(w, zb, (((1,), (0,)), ((), ())),
                            preferred_element_type=jnp.float32)
    z2 = jnp.sum(zb * zb, axis=0, keepdims=True)
    w2 = jnp.sum(w * w, axis=1, keepdims=True)
    d = (z2 + w2) - 2.0 * m

    minv = jnp.min(d, axis=0, keepdims=True)
    ci = jax.lax.broadcasted_iota(jnp.int32, (CODEBOOK, TOK), 0)
    big = jnp.int32(1 << 30)
    idx = jnp.min(jnp.where(d == minv, ci, big), axis=0, keepdims=True)
    idx_ref[0] = idx

    onehot = (ci == idx).astype(jnp.float32)
    zq = jax.lax.dot_general(wt_ref[...], onehot, (((1,), (0,)), ((), ())),
                             preferred_element_type=jnp.float32)
    zq_ref[0] = zb + (zq - zb)

    diff = zq - zb
    sq = jnp.sum(diff * diff)
    dsum = jnp.sum(d)
    cnt = jnp.sum(onehot, axis=1, keepdims=True)

    @pl.when(b == 0)
    def _init():
        dsum_acc[0, 0] = dsum
        sqsum_acc[0, 0] = sq
        counts_acc[...] = cnt

    @pl.when(b > 0)
    def _acc():
        dsum_acc[0, 0] += dsum
        sqsum_acc[0, 0] += sq
        counts_acc[...] += cnt

    @pl.when(b == B - 1)
    def _fin():
        md_ref[0, 0] = dsum_acc[0, 0] / jnp.float32(N_TOKENS * CODEBOOK)
        msq = sqsum_acc[0, 0] / jnp.float32(N_ELEMS)
        loss_ref[0, 0] = jnp.float32(BETA) * msq + msq
        e = counts_acc[...] / jnp.float32(N_TOKENS)
        ent = jnp.sum(e * jnp.log(e + jnp.float32(1e-10)))
        perp_ref[0, 0] = jnp.exp(-ent)


@functools.partial(jax.jit, static_argnames=("interpret",))
def kernel(z, W, interpret=False):
    z3 = z.reshape(B, EMB, TOK)
    wt = W.T
    grid = (B,)
    out_shapes = (
        jax.ShapeDtypeStruct((B, EMB, TOK), jnp.float32),
        jax.ShapeDtypeStruct((B, 1, TOK), jnp.int32),
        jax.ShapeDtypeStruct((1, 1), jnp.float32),
        jax.ShapeDtypeStruct((1, 1), jnp.float32),
        jax.ShapeDtypeStruct((1, 1), jnp.float32),
    )
    zq, idx, loss, perp, md = pl.pallas_call(
        _vq_body,
        grid=grid,
        in_specs=[
            pl.BlockSpec((1, EMB, TOK), lambda b: (b, 0, 0)),
            pl.BlockSpec((CODEBOOK, EMB), lambda b: (0, 0)),
            pl.BlockSpec((EMB, CODEBOOK), lambda b: (0, 0)),
        ],
        out_specs=(
            pl.BlockSpec((1, EMB, TOK), lambda b: (b, 0, 0)),
            pl.BlockSpec((1, 1, TOK), lambda b: (b, 0, 0)),
            pl.BlockSpec(memory_space=pltpu.SMEM),
            pl.BlockSpec(memory_space=pltpu.SMEM),
            pl.BlockSpec(memory_space=pltpu.SMEM),
        ),
        out_shape=out_shapes,
        scratch_shapes=[
            pltpu.SMEM((1, 1), jnp.float32),
            pltpu.SMEM((1, 1), jnp.float32),
            pltpu.VMEM((CODEBOOK, 1), jnp.float32),
        ],
        interpret=interpret,
    )(z3, W, wt)

    z_q = zq.reshape(B, EMB, 32, 32)
    min_encoding_indices = idx.reshape(N_TOKENS, 1)
    return (z_q, loss[0, 0], perp[0, 0], md[0, 0], min_encoding_indices)


# speedup vs baseline: 1.6538x; 1.0853x over previous
import functools

import jax
import jax.numpy as jnp
from jax.experimental import pallas as pl
from jax.experimental.pallas import tpu as pltpu

CODEBOOK = 1024
EMB = 256
B = 8
TOK = 1024
BETA = 0.25
N_TOKENS = B * TOK
N_ELEMS = N_TOKENS * EMB


def _vq_body(z_ref, w_ref,
             zq_ref, idx_ref, loss_ref, perp_ref, md_ref,
             dsum_acc, sqsum_acc, counts_acc):
    b = pl.program_id(0)
    zb = z_ref[0]
    w = w_ref[...]

    w2x = w + w
    m2 = jax.lax.dot_general(w2x, zb, (((1,), (0,)), ((), ())),
                             preferred_element_type=jnp.float32)
    z2 = jnp.sum(zb * zb, axis=0, keepdims=True)
    w2 = jnp.sum(w * w, axis=1, keepdims=True)
    d = (z2 + w2) - m2

    minv = jnp.min(d, axis=0, keepdims=True)
    ci = jax.lax.broadcasted_iota(jnp.int32, (CODEBOOK, TOK), 0)
    big = jnp.int32(1 << 30)
    idx = jnp.min(jnp.where(d == minv, ci, big), axis=0, keepdims=True)
    idx_ref[0] = idx

    onehot = (ci == idx).astype(jnp.float32)
    zq_ref[0] = jax.lax.dot_general(w, onehot, (((0,), (0,)), ((), ())),
                                    preferred_element_type=jnp.float32)

    wcs = jnp.sum(w2x, axis=0, keepdims=True)
    mrow = jax.lax.dot_general(wcs, zb, (((1,), (0,)), ((), ())),
                               preferred_element_type=jnp.float32)
    dsum = (jnp.float32(CODEBOOK) * (jnp.sum(z2) + jnp.sum(w2))
            - jnp.sum(mrow))
    sq = jnp.sum(minv)
    cnt = jnp.sum(onehot, axis=1, keepdims=True)

    @pl.when(b == 0)
    def _init():
        dsum_acc[0, 0] = dsum
        sqsum_acc[0, 0] = sq
        counts_acc[...] = cnt

    @pl.when(b > 0)
    def _acc():
        dsum_acc[0, 0] += dsum
        sqsum_acc[0, 0] += sq
        counts_acc[...] += cnt

    @pl.when(b == B - 1)
    def _fin():
        md_ref[0, 0] = dsum_acc[0, 0] / jnp.float32(N_TOKENS * CODEBOOK)
        msq = sqsum_acc[0, 0] / jnp.float32(N_ELEMS)
        loss_ref[0, 0] = jnp.float32(BETA) * msq + msq
        e = counts_acc[...] / jnp.float32(N_TOKENS)
        ent = jnp.sum(e * jnp.log(e + jnp.float32(1e-10)))
        perp_ref[0, 0] = jnp.exp(-ent)


@functools.partial(jax.jit, static_argnames=("interpret",))
def kernel(z, W, interpret=False):
    z3 = z.reshape(B, EMB, TOK)
    grid = (B,)
    out_shapes = (
        jax.ShapeDtypeStruct((B, EMB, TOK), jnp.float32),
        jax.ShapeDtypeStruct((B, 1, TOK), jnp.int32),
        jax.ShapeDtypeStruct((1, 1), jnp.float32),
        jax.ShapeDtypeStruct((1, 1), jnp.float32),
        jax.ShapeDtypeStruct((1, 1), jnp.float32),
    )
    zq, idx, loss, perp, md = pl.pallas_call(
        _vq_body,
        grid=grid,
        in_specs=[
            pl.BlockSpec((1, EMB, TOK), lambda b: (b, 0, 0)),
            pl.BlockSpec((CODEBOOK, EMB), lambda b: (0, 0)),
        ],
        out_specs=(
            pl.BlockSpec((1, EMB, TOK), lambda b: (b, 0, 0)),
            pl.BlockSpec((1, 1, TOK), lambda b: (b, 0, 0)),
            pl.BlockSpec(memory_space=pltpu.SMEM),
            pl.BlockSpec(memory_space=pltpu.SMEM),
            pl.BlockSpec(memory_space=pltpu.SMEM),
        ),
        out_shape=out_shapes,
        scratch_shapes=[
            pltpu.SMEM((1, 1), jnp.float32),
            pltpu.SMEM((1, 1), jnp.float32),
            pltpu.VMEM((CODEBOOK, 1), jnp.float32),
        ],
        interpret=interpret,
    )(z3, W)

    z_q = zq.reshape(B, EMB, 32, 32)
    min_encoding_indices = idx.reshape(N_TOKENS, 1)
    return (z_q, loss[0, 0], perp[0, 0], md[0, 0], min_encoding_indices)
